# Initial kernel scaffold; baseline (speedup 1.0000x reference)
import jax
import jax.numpy as jnp
from jax.experimental import pallas as pl


def kernel(x_num, x_cat, tables, W1, b1, W2, b2):
    raise NotImplementedError("write your pallas kernel here")


# speedup vs baseline: 1.4174x; 1.4174x over previous
import functools

import jax
import jax.numpy as jnp
from jax import lax
from jax.experimental import pallas as pl
from jax.experimental.pallas import tpu as pltpu
from jax.experimental.pallas import tpu_sc as plsc

B = 16384
F = 26
V = 100000
NNUM = 13
H = 128
OUT = 2

NC = 2
NS = 16
NW = NC * NS
N_LOOKUPS = B * F
N_PER_W = N_LOOKUPS // NW
CH = 128
NCH = N_PER_W // CH


def _gather_body(table_hbm, idx_hbm, out_hbm, idx_v, rows_v, sem):
    wid = lax.axis_index("s") * NC + lax.axis_index("c")
    pltpu.sync_copy(idx_hbm.at[wid], idx_v)

    def fire(j, _):
        pltpu.async_copy(table_hbm.at[idx_v.at[j]], rows_v.at[j], sem)
        return 0

    def drain(j, _):
        pltpu.make_async_copy(table_hbm.at[idx_v.at[j]], rows_v.at[j], sem).wait()
        return 0

    lax.fori_loop(0, NCH, fire, 0)
    lax.fori_loop(0, NCH, drain, 0)
    pltpu.sync_copy(rows_v, out_hbm.at[wid])


def _sc_gather(table_flat, idx3):
    mesh = plsc.VectorSubcoreMesh(core_axis_name="c", subcore_axis_name="s")
    run = pl.kernel(
        _gather_body,
        out_type=jax.ShapeDtypeStruct((NW, NCH, CH, 2), jnp.float32),
        mesh=mesh,
        scratch_types=[
            pltpu.VMEM((NCH, CH), jnp.int32),
            pltpu.VMEM((NCH, CH, 2), jnp.float32),
            pltpu.SemaphoreType.DMA,
        ],
        compiler_params=pltpu.CompilerParams(use_tc_tiling_on_sc=False),
    )
    return run(table_flat, idx3)


BB = 2048


def _mlp_body(xn_ref, xe_ref, w1n_ref, w1e_ref, b1_ref, w2_ref, b2_ref, o_ref):
    h = jnp.dot(xn_ref[...], w1n_ref[...], preferred_element_type=jnp.float32)
    h = h + jnp.dot(xe_ref[...], w1e_ref[...], preferred_element_type=jnp.float32)
    h = jnp.maximum(h + b1_ref[...], 0.0)
    y = jnp.dot(h, w2_ref[...], preferred_element_type=jnp.float32) + b2_ref[...]
    col = lax.broadcasted_iota(jnp.int32, y.shape, 1)
    o_ref[...] = jnp.where(col == 0, y, jnp.maximum(y, 0.0) + 0.025)


def _tc_mlp(x_num, x_emb, W1, b1, W2, b2):
    w1n = W1[:NNUM]
    w1e = W1[NNUM:]
    grid = (B // BB,)
    return pl.pallas_call(
        _mlp_body,
        grid=grid,
        in_specs=[
            pl.BlockSpec((BB, NNUM), lambda i: (i, 0)),
            pl.BlockSpec((BB, 2 * F), lambda i: (i, 0)),
            pl.BlockSpec((NNUM, H), lambda i: (0, 0)),
            pl.BlockSpec((2 * F, H), lambda i: (0, 0)),
            pl.BlockSpec((1, H), lambda i: (0, 0)),
            pl.BlockSpec((H, OUT), lambda i: (0, 0)),
            pl.BlockSpec((1, OUT), lambda i: (0, 0)),
        ],
        out_specs=pl.BlockSpec((BB, OUT), lambda i: (i, 0)),
        out_shape=jax.ShapeDtypeStruct((B, OUT), jnp.float32),
    )(x_num, x_emb, w1n, w1e, b1.reshape(1, H), W2, b2.reshape(1, OUT))


def kernel(x_num, x_cat, tables, W1, b1, W2, b2):
    table_flat = tables.reshape(F * V, 2)
    flat_idx = (x_cat + (jnp.arange(F, dtype=jnp.int32) * V)[None, :])
    idx3 = flat_idx.reshape(NW, NCH, CH)
    emb = _sc_gather(table_flat, idx3)
    x_emb = emb.reshape(B, 2 * F)
    return _tc_mlp(x_num, x_emb, W1, b1, W2, b2)


# speedup vs baseline: 1.7850x; 1.2593x over previous
import functools

import jax
import jax.numpy as jnp
from jax import lax
from jax.experimental import pallas as pl
from jax.experimental.pallas import tpu as pltpu
from jax.experimental.pallas import tpu_sc as plsc

B = 16384
F = 26
V = 100000
NNUM = 13
H = 128
OUT = 2

NC = 2
NS = 16
NW = NC * NS
N_WORDS = B * F * 2
W_PER_T = N_WORDS // NW
CH = 128
NCH = W_PER_T // CH


def _gather_body(table_hbm, idx_hbm, out_hbm, idx_v, rows_v, sem):
    wid = lax.axis_index("s") * NC + lax.axis_index("c")
    pltpu.sync_copy(idx_hbm.at[wid], idx_v)

    def fire(j, _):
        pltpu.async_copy(table_hbm.at[idx_v.at[j]], rows_v.at[j], sem)
        return 0

    def drain(j, _):
        pltpu.make_async_copy(table_hbm.at[idx_v.at[j]], rows_v.at[j], sem).wait()
        return 0

    lax.fori_loop(0, NCH, fire, 0)
    lax.fori_loop(0, NCH, drain, 0)
    pltpu.sync_copy(rows_v, out_hbm.at[wid])


def _sc_gather(table_1d, idx3):
    mesh = plsc.VectorSubcoreMesh(core_axis_name="c", subcore_axis_name="s")
    run = pl.kernel(
        _gather_body,
        out_type=jax.ShapeDtypeStruct((NW, NCH, CH), jnp.float32),
        mesh=mesh,
        scratch_types=[
            pltpu.VMEM((NCH, CH), jnp.int32),
            pltpu.VMEM((NCH, CH), jnp.float32),
            pltpu.SemaphoreType.DMA,
        ],
        compiler_params=pltpu.CompilerParams(use_tc_tiling_on_sc=False),
    )
    return run(table_1d, idx3)


BB = 2048


def _mlp_body(xn_ref, xe_ref, w1n_ref, w1e_ref, b1_ref, w2_ref, b2_ref, o_ref):
    h = jnp.dot(xn_ref[...], w1n_ref[...], preferred_element_type=jnp.float32)
    h = h + jnp.dot(xe_ref[...], w1e_ref[...], preferred_element_type=jnp.float32)
    h = jnp.maximum(h + b1_ref[...], 0.0)
    y = jnp.dot(h, w2_ref[...], preferred_element_type=jnp.float32) + b2_ref[...]
    col = lax.broadcasted_iota(jnp.int32, y.shape, 1)
    o_ref[...] = jnp.where(col == 0, y, jnp.maximum(y, 0.0) + 0.025)


def _tc_mlp(x_num, x_emb, W1, b1, W2, b2):
    w1n = W1[:NNUM]
    w1e = W1[NNUM:]
    grid = (B // BB,)
    return pl.pallas_call(
        _mlp_body,
        grid=grid,
        in_specs=[
            pl.BlockSpec((BB, NNUM), lambda i: (i, 0)),
            pl.BlockSpec((BB, 2 * F), lambda i: (i, 0)),
            pl.BlockSpec((NNUM, H), lambda i: (0, 0)),
            pl.BlockSpec((2 * F, H), lambda i: (0, 0)),
            pl.BlockSpec((1, H), lambda i: (0, 0)),
            pl.BlockSpec((H, OUT), lambda i: (0, 0)),
            pl.BlockSpec((1, OUT), lambda i: (0, 0)),
        ],
        out_specs=pl.BlockSpec((BB, OUT), lambda i: (i, 0)),
        out_shape=jax.ShapeDtypeStruct((B, OUT), jnp.float32),
    )(x_num, x_emb, w1n, w1e, b1.reshape(1, H), W2, b2.reshape(1, OUT))


def kernel(x_num, x_cat, tables, W1, b1, W2, b2):
    table_1d = tables.reshape(F * V * 2)
    base = (x_cat + (jnp.arange(F, dtype=jnp.int32) * V)[None, :]) * 2
    idx_pairs = jnp.stack([base, base + 1], axis=-1)
    idx3 = idx_pairs.reshape(NW, NCH, CH)
    emb = _sc_gather(table_1d, idx3)
    x_emb = emb.reshape(B, 2 * F)
    return _tc_mlp(x_num, x_emb, W1, b1, W2, b2)


# speedup vs baseline: 41.8752x; 23.4596x over previous
import functools

import jax
import jax.numpy as jnp
from jax import lax
from jax.experimental import pallas as pl
from jax.experimental.pallas import tpu as pltpu
from jax.experimental.pallas import tpu_sc as plsc

B = 16384
F = 26
V = 100000
NNUM = 13
H = 128
OUT = 2

NC = 2
NS = 16
NW = NC * NS
N_WORDS = B * F * 2
W_PER_T = N_WORDS // NW
CH = 128
NCH = W_PER_T // CH


def _gather_body(table_hbm, idx_hbm, out_hbm, idx_v, rows_v, sem):
    wid = lax.axis_index("s") * NC + lax.axis_index("c")
    pltpu.sync_copy(idx_hbm.at[wid], idx_v)

    def fire(j, _):
        pltpu.async_copy(table_hbm.at[idx_v.at[j]], rows_v.at[j], sem)
        return 0

    def drain(j, _):
        pltpu.make_async_copy(table_hbm.at[idx_v.at[j]], rows_v.at[j], sem).wait()
        return 0

    lax.fori_loop(0, NCH, fire, 0)
    lax.fori_loop(0, NCH, drain, 0)
    pltpu.sync_copy(rows_v, out_hbm.at[wid])


def _sc_gather(table_1d, idx3):
    mesh = plsc.VectorSubcoreMesh(core_axis_name="c", subcore_axis_name="s")
    run = pl.kernel(
        _gather_body,
        out_type=jax.ShapeDtypeStruct((NW, NCH, CH), jnp.float32),
        mesh=mesh,
        scratch_types=[
            pltpu.VMEM((NCH, CH), jnp.int32),
            pltpu.VMEM((NCH, CH), jnp.float32),
            pltpu.SemaphoreType.DMA,
        ],
        compiler_params=pltpu.CompilerParams(use_tc_tiling_on_sc=False),
    )
    return run(table_1d, idx3)


BB = 2048


def _mlp_body(xn_ref, xe_ref, w1n_ref, w1e_ref, b1_ref, w2_ref, b2_ref, o_ref):
    h = jnp.dot(xn_ref[...], w1n_ref[...], preferred_element_type=jnp.float32)
    h = h + jnp.dot(xe_ref[...], w1e_ref[...], preferred_element_type=jnp.float32)
    h = jnp.maximum(h + b1_ref[...], 0.0)
    y = jnp.dot(h, w2_ref[...], preferred_element_type=jnp.float32) + b2_ref[...]
    col = lax.broadcasted_iota(jnp.int32, y.shape, 1)
    o_ref[...] = jnp.where(col == 0, y, jnp.maximum(y, 0.0) + 0.025)


def _tc_mlp(x_num, x_emb, W1, b1, W2, b2):
    w1n = W1[:NNUM]
    w1e = W1[NNUM:]
    grid = (B // BB,)
    return pl.pallas_call(
        _mlp_body,
        grid=grid,
        in_specs=[
            pl.BlockSpec((BB, NNUM), lambda i: (i, 0)),
            pl.BlockSpec((BB, 2 * F), lambda i: (i, 0)),
            pl.BlockSpec((NNUM, H), lambda i: (0, 0)),
            pl.BlockSpec((2 * F, H), lambda i: (0, 0)),
            pl.BlockSpec((1, H), lambda i: (0, 0)),
            pl.BlockSpec((H, OUT), lambda i: (0, 0)),
            pl.BlockSpec((1, OUT), lambda i: (0, 0)),
        ],
        out_specs=pl.BlockSpec((BB, OUT), lambda i: (i, 0)),
        out_shape=jax.ShapeDtypeStruct((B, OUT), jnp.float32),
    )(x_num, x_emb, w1n, w1e, b1.reshape(1, H), W2, b2.reshape(1, OUT))


def kernel(x_num, x_cat, tables, W1, b1, W2, b2):
    table_1d = tables.transpose(0, 2, 1).reshape(F * 2 * V)
    cols = jnp.arange(2 * F, dtype=jnp.int32)
    idx52 = jnp.take(x_cat, cols // 2, axis=1) + cols[None, :] * V
    idx3 = idx52.reshape(NW, NCH, CH)
    emb = _sc_gather(table_1d, idx3)
    x_emb = emb.reshape(B, 2 * F)
    return _tc_mlp(x_num, x_emb, W1, b1, W2, b2)


# speedup vs baseline: 41.9025x; 1.0007x over previous
import functools

import jax
import jax.numpy as jnp
from jax import lax
from jax.experimental import pallas as pl
from jax.experimental.pallas import tpu as pltpu
from jax.experimental.pallas import tpu_sc as plsc

B = 16384
F = 26
V = 100000
NNUM = 13
H = 128
OUT = 2

NC = 2
NS = 16
NW = NC * NS
N_WORDS = B * F * 2
W_PER_T = N_WORDS // NW
CH = 128
NCH = W_PER_T // CH


def _gather_body(table_hbm, idx_hbm, out_hbm, idx_v, rows_v, sem, sem_out):
    wid = lax.axis_index("s") * NC + lax.axis_index("c")
    pltpu.sync_copy(idx_hbm.at[wid], idx_v)

    def fire(j, _):
        pltpu.async_copy(table_hbm.at[idx_v.at[j]], rows_v.at[j], sem)
        return 0

    def drain_and_store(j, _):
        pltpu.make_async_copy(table_hbm.at[idx_v.at[j]], rows_v.at[j], sem).wait()
        pltpu.async_copy(rows_v.at[j], out_hbm.at[wid].at[j], sem_out)
        return 0

    def drain_out(j, _):
        pltpu.make_async_copy(rows_v.at[j], out_hbm.at[wid].at[j], sem_out).wait()
        return 0

    lax.fori_loop(0, NCH, fire, 0)
    lax.fori_loop(0, NCH, drain_and_store, 0)
    lax.fori_loop(0, NCH, drain_out, 0)


def _sc_gather(table_1d, idx3):
    mesh = plsc.VectorSubcoreMesh(core_axis_name="c", subcore_axis_name="s")
    run = pl.kernel(
        _gather_body,
        out_type=jax.ShapeDtypeStruct((NW, NCH, CH), jnp.float32),
        mesh=mesh,
        scratch_types=[
            pltpu.VMEM((NCH, CH), jnp.int32),
            pltpu.VMEM((NCH, CH), jnp.float32),
            pltpu.SemaphoreType.DMA,
            pltpu.SemaphoreType.DMA,
        ],
        compiler_params=pltpu.CompilerParams(use_tc_tiling_on_sc=False),
    )
    return run(table_1d, idx3)


BB = 2048


def _mlp_body(xn_ref, xe_ref, w1n_ref, w1e_ref, b1_ref, w2_ref, b2_ref, o_ref):
    h = jnp.dot(xn_ref[...], w1n_ref[...], preferred_element_type=jnp.float32)
    h = h + jnp.dot(xe_ref[...], w1e_ref[...], preferred_element_type=jnp.float32)
    h = jnp.maximum(h + b1_ref[...], 0.0)
    y = jnp.dot(h, w2_ref[...], preferred_element_type=jnp.float32) + b2_ref[...]
    col = lax.broadcasted_iota(jnp.int32, y.shape, 1)
    o_ref[...] = jnp.where(col == 0, y, jnp.maximum(y, 0.0) + 0.025)


def _tc_mlp(x_num, x_emb, W1, b1, W2, b2):
    w1n = W1[:NNUM]
    w1e = W1[NNUM:]
    grid = (B // BB,)
    return pl.pallas_call(
        _mlp_body,
        grid=grid,
        in_specs=[
            pl.BlockSpec((BB, NNUM), lambda i: (i, 0)),
            pl.BlockSpec((BB, 2 * F), lambda i: (i, 0)),
            pl.BlockSpec((NNUM, H), lambda i: (0, 0)),
            pl.BlockSpec((2 * F, H), lambda i: (0, 0)),
            pl.BlockSpec((1, H), lambda i: (0, 0)),
            pl.BlockSpec((H, OUT), lambda i: (0, 0)),
            pl.BlockSpec((1, OUT), lambda i: (0, 0)),
        ],
        out_specs=pl.BlockSpec((BB, OUT), lambda i: (i, 0)),
        out_shape=jax.ShapeDtypeStruct((B, OUT), jnp.float32),
    )(x_num, x_emb, w1n, w1e, b1.reshape(1, H), W2, b2.reshape(1, OUT))


def kernel(x_num, x_cat, tables, W1, b1, W2, b2):
    table_1d = tables.transpose(0, 2, 1).reshape(F * 2 * V)
    cols = jnp.arange(2 * F, dtype=jnp.int32)
    idx52 = jnp.take(x_cat, cols // 2, axis=1) + cols[None, :] * V
    idx3 = idx52.reshape(NW, NCH, CH)
    emb = _sc_gather(table_1d, idx3)
    x_emb = emb.reshape(B, 2 * F)
    return _tc_mlp(x_num, x_emb, W1, b1, W2, b2)


# speedup vs baseline: 43.0569x; 1.0275x over previous
import functools

import jax
import jax.numpy as jnp
from jax import lax
from jax.experimental import pallas as pl
from jax.experimental.pallas import tpu as pltpu
from jax.experimental.pallas import tpu_sc as plsc

B = 16384
F = 26
V = 100000
NNUM = 13
H = 128
OUT = 2

NC = 2
NS = 16
NW = NC * NS
N_WORDS = B * F * 2
W_PER_T = N_WORDS // NW
CH = 128
NCH = W_PER_T // CH


def _gather_body(table_hbm, idx_hbm, out_hbm, idx_v, rows_v, sem, sem_out):
    wid = lax.axis_index("s") * NC + lax.axis_index("c")
    pltpu.sync_copy(idx_hbm.at[wid], idx_v)

    def fire(j, _):
        pltpu.async_copy(table_hbm.at[idx_v.at[j]], rows_v.at[j], sem)
        return 0

    def drain_and_store(j, _):
        pltpu.make_async_copy(table_hbm.at[idx_v.at[j]], rows_v.at[j], sem).wait()
        pltpu.async_copy(rows_v.at[j], out_hbm.at[wid].at[j], sem_out)
        return 0

    def drain_out(j, _):
        pltpu.make_async_copy(rows_v.at[j], out_hbm.at[wid].at[j], sem_out).wait()
        return 0

    lax.fori_loop(0, NCH, fire, 0)
    lax.fori_loop(0, NCH, drain_and_store, 0)
    lax.fori_loop(0, NCH, drain_out, 0)


def _sc_gather(table_1d, idx3):
    mesh = plsc.VectorSubcoreMesh(core_axis_name="c", subcore_axis_name="s")
    run = pl.kernel(
        _gather_body,
        out_type=jax.ShapeDtypeStruct((NW, NCH, CH), jnp.float32),
        mesh=mesh,
        scratch_types=[
            pltpu.VMEM((NCH, CH), jnp.int32),
            pltpu.VMEM((NCH, CH), jnp.float32),
            pltpu.SemaphoreType.DMA,
            pltpu.SemaphoreType.DMA,
        ],
        compiler_params=pltpu.CompilerParams(use_tc_tiling_on_sc=False),
    )
    return run(table_1d, idx3)


BB = 2048


def _mlp_body(xn_ref, xe_ref, w1n_ref, w1e_ref, b1_ref, w2_ref, b2_ref, o_ref):
    h = lax.dot_general(xn_ref[...], w1n_ref[...], (((0,), (0,)), ((), ())),
                        preferred_element_type=jnp.float32)
    h = h + jnp.dot(xe_ref[...], w1e_ref[...], preferred_element_type=jnp.float32)
    h = jnp.maximum(h + b1_ref[...], 0.0)
    y = jnp.dot(h, w2_ref[...], preferred_element_type=jnp.float32) + b2_ref[...]
    col = lax.broadcasted_iota(jnp.int32, y.shape, 1)
    o_ref[...] = jnp.where(col == 0, y, jnp.maximum(y, 0.0) + 0.025)


def _tc_mlp(x_num, x_emb, W1, b1, W2, b2):
    w1n = W1[:NNUM]
    w1e = W1[NNUM:]
    grid = (B // BB,)
    return pl.pallas_call(
        _mlp_body,
        grid=grid,
        in_specs=[
            pl.BlockSpec((NNUM, BB), lambda i: (0, i)),
            pl.BlockSpec((BB, 2 * F), lambda i: (i, 0)),
            pl.BlockSpec((NNUM, H), lambda i: (0, 0)),
            pl.BlockSpec((2 * F, H), lambda i: (0, 0)),
            pl.BlockSpec((1, H), lambda i: (0, 0)),
            pl.BlockSpec((H, OUT), lambda i: (0, 0)),
            pl.BlockSpec((1, OUT), lambda i: (0, 0)),
        ],
        out_specs=pl.BlockSpec((BB, OUT), lambda i: (i, 0)),
        out_shape=jax.ShapeDtypeStruct((B, OUT), jnp.float32),
    )(x_num.T, x_emb, w1n, w1e, b1.reshape(1, H), W2, b2.reshape(1, OUT))


def kernel(x_num, x_cat, tables, W1, b1, W2, b2):
    table_1d = tables.transpose(0, 2, 1).reshape(F * 2 * V)
    cols = jnp.arange(2 * F, dtype=jnp.int32)
    idx52 = jnp.take(x_cat, cols // 2, axis=1) + cols[None, :] * V
    idx3 = idx52.reshape(NW, NCH, CH)
    emb = _sc_gather(table_1d, idx3)
    x_emb = emb.reshape(B, 2 * F)
    return _tc_mlp(x_num, x_emb, W1, b1, W2, b2)


# speedup vs baseline: 51.1144x; 1.1871x over previous
import functools

import jax
import jax.numpy as jnp
from jax import lax
from jax.experimental import pallas as pl
from jax.experimental.pallas import tpu as pltpu
from jax.experimental.pallas import tpu_sc as plsc

B = 16384
F = 26
V = 100000
NNUM = 13
H = 128
OUT = 2

NC = 2
NS = 16
NW = NC * NS
N_WORDS = B * F * 2
W_PER_T = N_WORDS // NW
CH = 128
NCH = W_PER_T // CH


def _gather_body(table_hbm, idx_hbm, out_hbm, idx_v, rows_v, sem, sem_out):
    wid = lax.axis_index("s") * NC + lax.axis_index("c")
    pltpu.sync_copy(idx_hbm.at[wid], idx_v)

    def fire(j, _):
        pltpu.async_copy(table_hbm.at[idx_v.at[j]], rows_v.at[j], sem)
        return 0

    def drain_and_store(j, _):
        pltpu.make_async_copy(table_hbm.at[idx_v.at[j]], rows_v.at[j], sem).wait()
        pltpu.async_copy(rows_v.at[j], out_hbm.at[wid].at[j], sem_out)
        return 0

    def drain_out(j, _):
        pltpu.make_async_copy(rows_v.at[j], out_hbm.at[wid].at[j], sem_out).wait()
        return 0

    lax.fori_loop(0, NCH, fire, 0)
    lax.fori_loop(0, NCH, drain_and_store, 0)
    lax.fori_loop(0, NCH, drain_out, 0)


def _sc_gather(table_1d, idx3):
    mesh = plsc.VectorSubcoreMesh(core_axis_name="c", subcore_axis_name="s")
    run = pl.kernel(
        _gather_body,
        out_type=jax.ShapeDtypeStruct((NW, NCH, CH), jnp.float32),
        mesh=mesh,
        scratch_types=[
            pltpu.VMEM((NCH, CH), jnp.int32),
            pltpu.VMEM((NCH, CH), jnp.float32),
            pltpu.SemaphoreType.DMA,
            pltpu.SemaphoreType.DMA,
        ],
        compiler_params=pltpu.CompilerParams(use_tc_tiling_on_sc=False),
    )
    return run(table_1d, idx3)


BB = 2048


def _mlp_body(xn_ref, xe_ref, w1n_ref, w1e_ref, b1_ref, w2_ref, b2_ref, o_ref):
    h = lax.dot_general(xn_ref[...], w1n_ref[...], (((0,), (0,)), ((), ())),
                        preferred_element_type=jnp.float32)
    h = h + lax.dot_general(xe_ref[...], w1e_ref[...], (((0,), (0,)), ((), ())),
                            preferred_element_type=jnp.float32)
    h = jnp.maximum(h + b1_ref[...], 0.0)
    y = jnp.dot(h, w2_ref[...], preferred_element_type=jnp.float32) + b2_ref[...]
    col = lax.broadcasted_iota(jnp.int32, y.shape, 1)
    o_ref[...] = jnp.where(col == 0, y, jnp.maximum(y, 0.0) + 0.025)


def _tc_mlp(x_num, x_emb, W1, b1, W2, b2):
    w1n = W1[:NNUM]
    w1e = W1[NNUM:]
    grid = (B // BB,)
    return pl.pallas_call(
        _mlp_body,
        grid=grid,
        in_specs=[
            pl.BlockSpec((NNUM, BB), lambda i: (0, i)),
            pl.BlockSpec((2 * F, BB), lambda i: (0, i)),
            pl.BlockSpec((NNUM, H), lambda i: (0, 0)),
            pl.BlockSpec((2 * F, H), lambda i: (0, 0)),
            pl.BlockSpec((1, H), lambda i: (0, 0)),
            pl.BlockSpec((H, OUT), lambda i: (0, 0)),
            pl.BlockSpec((1, OUT), lambda i: (0, 0)),
        ],
        out_specs=pl.BlockSpec((BB, OUT), lambda i: (i, 0)),
        out_shape=jax.ShapeDtypeStruct((B, OUT), jnp.float32),
    )(x_num.T, x_emb, w1n, w1e, b1.reshape(1, H), W2, b2.reshape(1, OUT))


def kernel(x_num, x_cat, tables, W1, b1, W2, b2):
    table_1d = tables.transpose(0, 2, 1).reshape(F * 2 * V)
    cols = jnp.arange(2 * F, dtype=jnp.int32)
    idxT = jnp.take(x_cat.T, cols // 2, axis=0) + cols[:, None] * V
    idx3 = idxT.reshape(NW, NCH, CH)
    emb = _sc_gather(table_1d, idx3)
    x_emb_t = emb.reshape(2 * F, B)
    return _tc_mlp(x_num, x_emb_t, W1, b1, W2, b2)


# speedup vs baseline: 52.3084x; 1.0234x over previous
import functools

import jax
import jax.numpy as jnp
from jax import lax
from jax.experimental import pallas as pl
from jax.experimental.pallas import tpu as pltpu
from jax.experimental.pallas import tpu_sc as plsc

B = 16384
F = 26
V = 100000
NNUM = 13
H = 128
OUT = 2

NC = 2
NS = 16
NW = NC * NS
N_WORDS = B * F * 2
W_PER_T = N_WORDS // NW
CH = 128
NCH = W_PER_T // CH


def _gather_body(table_hbm, idx_hbm, out_hbm, idx_v, rows_v, sem, sem_out, sem_idx):
    wid = lax.axis_index("s") * NC + lax.axis_index("c")

    def fire_idx(j, _):
        pltpu.async_copy(idx_hbm.at[wid].at[j], idx_v.at[j], sem_idx)
        return 0

    def fire(j, _):
        pltpu.make_async_copy(idx_hbm.at[wid].at[j], idx_v.at[j], sem_idx).wait()
        pltpu.async_copy(table_hbm.at[idx_v.at[j]], rows_v.at[j], sem)
        return 0

    def drain_and_store(j, _):
        pltpu.make_async_copy(table_hbm.at[idx_v.at[j]], rows_v.at[j], sem).wait()
        pltpu.async_copy(rows_v.at[j], out_hbm.at[wid].at[j], sem_out)
        return 0

    def drain_out(j, _):
        pltpu.make_async_copy(rows_v.at[j], out_hbm.at[wid].at[j], sem_out).wait()
        return 0

    lax.fori_loop(0, NCH, fire_idx, 0)
    lax.fori_loop(0, NCH, fire, 0)
    lax.fori_loop(0, NCH, drain_and_store, 0)
    lax.fori_loop(0, NCH, drain_out, 0)


def _sc_gather(table_1d, idx3):
    mesh = plsc.VectorSubcoreMesh(core_axis_name="c", subcore_axis_name="s")
    run = pl.kernel(
        _gather_body,
        out_type=jax.ShapeDtypeStruct((NW, NCH, CH), jnp.float32),
        mesh=mesh,
        scratch_types=[
            pltpu.VMEM((NCH, CH), jnp.int32),
            pltpu.VMEM((NCH, CH), jnp.float32),
            pltpu.SemaphoreType.DMA,
            pltpu.SemaphoreType.DMA,
            pltpu.SemaphoreType.DMA,
        ],
        compiler_params=pltpu.CompilerParams(use_tc_tiling_on_sc=False),
    )
    return run(table_1d, idx3)


BB = 4096


def _mlp_body(xn_ref, xe_ref, w1n_ref, w1e_ref, b1_ref, w2_ref, b2_ref, o_ref):
    h = lax.dot_general(xn_ref[...], w1n_ref[...], (((0,), (0,)), ((), ())),
                        preferred_element_type=jnp.float32)
    h = h + lax.dot_general(xe_ref[...], w1e_ref[...], (((0,), (0,)), ((), ())),
                            preferred_element_type=jnp.float32)
    h = jnp.maximum(h + b1_ref[...], 0.0)
    y = jnp.dot(h, w2_ref[...], preferred_element_type=jnp.float32) + b2_ref[...]
    col = lax.broadcasted_iota(jnp.int32, y.shape, 1)
    o_ref[...] = jnp.where(col == 0, y, jnp.maximum(y, 0.0) + 0.025)


def _tc_mlp(x_num, x_emb, W1, b1, W2, b2):
    w1n = W1[:NNUM]
    w1e = W1[NNUM:]
    grid = (B // BB,)
    return pl.pallas_call(
        _mlp_body,
        grid=grid,
        in_specs=[
            pl.BlockSpec((NNUM, BB), lambda i: (0, i)),
            pl.BlockSpec((2 * F, BB), lambda i: (0, i)),
            pl.BlockSpec((NNUM, H), lambda i: (0, 0)),
            pl.BlockSpec((2 * F, H), lambda i: (0, 0)),
            pl.BlockSpec((1, H), lambda i: (0, 0)),
            pl.BlockSpec((H, OUT), lambda i: (0, 0)),
            pl.BlockSpec((1, OUT), lambda i: (0, 0)),
        ],
        out_specs=pl.BlockSpec((BB, OUT), lambda i: (i, 0)),
        out_shape=jax.ShapeDtypeStruct((B, OUT), jnp.float32),
    )(x_num.T, x_emb, w1n, w1e, b1.reshape(1, H), W2, b2.reshape(1, OUT))


def kernel(x_num, x_cat, tables, W1, b1, W2, b2):
    table_1d = tables.transpose(0, 2, 1).reshape(F * 2 * V)
    cols = jnp.arange(2 * F, dtype=jnp.int32)
    idxT = jnp.take(x_cat.T, cols // 2, axis=0) + cols[:, None] * V
    idx3 = idxT.reshape(NW, NCH, CH)
    emb = _sc_gather(table_1d, idx3)
    x_emb_t = emb.reshape(2 * F, B)
    return _tc_mlp(x_num, x_emb_t, W1, b1, W2, b2)
